# Initial kernel scaffold; baseline (speedup 1.0000x reference)
import jax
import jax.numpy as jnp
from jax.experimental import pallas as pl


def kernel(tokens, Wr, W1, b1, W2, b2):
    raise NotImplementedError("write your pallas kernel here")


# speedup vs baseline: 2.8831x; 2.8831x over previous
import functools

import jax
import jax.numpy as jnp
from jax import lax
from jax.experimental import pallas as pl
from jax.experimental.pallas import tpu as pltpu

HIDDEN = 1024
NUM_EXPERTS = 8
N_TOKENS = 4096
TOKEN_BLOCK = 512
N_TB = N_TOKENS // TOKEN_BLOCK


def _gates_for_expert(x, wr, e):
    logits = lax.dot_general(x, wr, (((1,), (1,)), ((), ())),
                             preferred_element_type=jnp.float32)
    idx = lax.broadcasted_iota(jnp.int32, logits.shape, 1)
    m1 = jnp.max(logits, axis=-1, keepdims=True)
    a1 = jnp.min(jnp.where(logits == m1, idx, NUM_EXPERTS), axis=-1,
                 keepdims=True)
    neg = jnp.float32(-jnp.inf)
    l2 = jnp.where(idx == a1, neg, logits)
    m2 = jnp.max(l2, axis=-1, keepdims=True)
    a2 = jnp.min(jnp.where(l2 == m2, idx, NUM_EXPERTS), axis=-1,
                 keepdims=True)
    p1 = 1.0 / (1.0 + jnp.exp(m2 - m1))
    p2 = 1.0 - p1
    return p1 * (a1 == e) + p2 * (a2 == e)


def _gelu_exact(x):
    return x * 0.5 * (1.0 + lax.erf(x * 0.7071067811865476))


def _moe_body(tokens_ref, wr_ref, w1_ref, b1_ref, w2_ref, b2_ref, out_ref):
    e = pl.program_id(1)
    x = tokens_ref[...]
    gate = _gates_for_expert(x, wr_ref[...], e)
    h = lax.dot_general(x, w1_ref[0], (((1,), (1,)), ((), ())),
                        preferred_element_type=jnp.float32)
    h = _gelu_exact(h + b1_ref[0])
    y = lax.dot_general(h, w2_ref[0], (((1,), (1,)), ((), ())),
                        preferred_element_type=jnp.float32)
    y = (y + b2_ref[0]) * gate

    @pl.when(e == 0)
    def _init():
        out_ref[...] = y

    @pl.when(e != 0)
    def _acc():
        out_ref[...] += y


@jax.jit
def kernel(tokens, Wr, W1, b1, W2, b2):
    grid = (N_TB, NUM_EXPERTS)
    return pl.pallas_call(
        _moe_body,
        grid=grid,
        in_specs=[
            pl.BlockSpec((TOKEN_BLOCK, HIDDEN), lambda t, e: (t, 0)),
            pl.BlockSpec((NUM_EXPERTS, HIDDEN), lambda t, e: (0, 0)),
            pl.BlockSpec((1, 2 * HIDDEN, HIDDEN), lambda t, e: (e, 0, 0)),
            pl.BlockSpec((1, 1, 2 * HIDDEN), lambda t, e: (e, 0, 0)),
            pl.BlockSpec((1, HIDDEN, 2 * HIDDEN), lambda t, e: (e, 0, 0)),
            pl.BlockSpec((1, 1, HIDDEN), lambda t, e: (e, 0, 0)),
        ],
        out_specs=pl.BlockSpec((TOKEN_BLOCK, HIDDEN), lambda t, e: (t, 0)),
        out_shape=jax.ShapeDtypeStruct((N_TOKENS, HIDDEN), jnp.float32),
    )(tokens, Wr, W1, b1[:, None, :], W2, b2[:, None, :])


# speedup vs baseline: 4.0240x; 1.3957x over previous
import functools

import jax
import jax.numpy as jnp
from jax import lax
from jax.experimental import pallas as pl
from jax.experimental.pallas import tpu as pltpu
from jax.experimental.pallas import tpu_sc as plsc

HIDDEN = 1024
FF = 2 * HIDDEN
NUM_EXPERTS = 8
N_TOKENS = 4096
TOP_K = 2
BLK = 256
NBLK = 40
S_PAD = NBLK * BLK
CHUNK = 512

NW = 32
TPW = N_TOKENS // NW


def _gelu_exact(x):
    return x * 0.5 * (1.0 + lax.erf(x * 0.7071067811865476))


def _meta_body(tokens_ref, wr_ref, slot_ref, p_ref, be_ref):
    x = tokens_ref[...]
    logits = lax.dot_general(x, wr_ref[...], (((1,), (1,)), ((), ())),
                             preferred_element_type=jnp.float32)
    idx8 = lax.broadcasted_iota(jnp.int32, logits.shape, 1)
    m1 = jnp.max(logits, axis=-1, keepdims=True)
    a1 = jnp.min(jnp.where(logits == m1, idx8, NUM_EXPERTS), axis=-1,
                 keepdims=True)
    l2 = jnp.where(idx8 == a1, jnp.float32(-1e30), logits)
    m2 = jnp.max(l2, axis=-1, keepdims=True)
    a2 = jnp.min(jnp.where(l2 == m2, idx8, NUM_EXPERTS), axis=-1,
                 keepdims=True)
    p1 = 1.0 / (1.0 + jnp.exp(m2 - m1))
    p2 = 1.0 - p1

    oh = (idx8 == a1).astype(jnp.float32) + (idx8 == a2).astype(jnp.float32)
    run = jnp.zeros((1, NUM_EXPERTS), jnp.float32)
    cum_chunks = []
    ri = lax.broadcasted_iota(jnp.int32, (CHUNK, CHUNK), 0)
    ci = lax.broadcasted_iota(jnp.int32, (CHUNK, CHUNK), 1)
    lstrict = (ci < ri).astype(jnp.float32)
    for c in range(N_TOKENS // CHUNK):
        blk = oh[c * CHUNK:(c + 1) * CHUNK]
        cum_chunks.append(
            lax.dot_general(lstrict, blk, (((1,), (0,)), ((), ())),
                            preferred_element_type=jnp.float32) + run)
        run = run + jnp.sum(blk, axis=0, keepdims=True)
    cum = jnp.concatenate(cum_chunks, axis=0)
    counts = run

    nblk_e = jnp.ceil(counts * (1.0 / BLK))
    er = lax.broadcasted_iota(jnp.int32, (NUM_EXPERTS, NUM_EXPERTS), 0)
    ec = lax.broadcasted_iota(jnp.int32, (NUM_EXPERTS, NUM_EXPERTS), 1)
    estrict = (er < ec).astype(jnp.float32)
    base_blk = lax.dot_general(nblk_e, estrict, (((1,), (0,)), ((), ())),
                               preferred_element_type=jnp.float32)
    base = base_blk * float(BLK)

    base_bc = jnp.broadcast_to(base, (N_TOKENS, NUM_EXPERTS))
    slot0 = jnp.sum(jnp.where(idx8 == a1, base_bc + cum, 0.0), axis=-1,
                    keepdims=True)
    slot1 = jnp.sum(jnp.where(idx8 == a2, base_bc + cum, 0.0), axis=-1,
                    keepdims=True)
    slot_ref[...] = jnp.concatenate([slot0, slot1], axis=1).astype(jnp.int32)
    p_ref[...] = jnp.concatenate([p1, p2], axis=1)

    bi = lax.broadcasted_iota(jnp.int32, (NBLK, NUM_EXPERTS), 0).astype(
        jnp.float32)
    cond = (jnp.broadcast_to(base_blk, (NBLK, NUM_EXPERTS)) <= bi)
    be_ref[...] = (jnp.sum(cond.astype(jnp.float32), axis=-1, keepdims=True)
                   - 1.0).astype(jnp.int32)


def _route_meta(tokens, Wr):
    return pl.pallas_call(
        _meta_body,
        in_specs=[
            pl.BlockSpec((N_TOKENS, HIDDEN), lambda: (0, 0)),
            pl.BlockSpec((NUM_EXPERTS, HIDDEN), lambda: (0, 0)),
        ],
        out_specs=[
            pl.BlockSpec((N_TOKENS, TOP_K), lambda: (0, 0)),
            pl.BlockSpec((N_TOKENS, TOP_K), lambda: (0, 0)),
            pl.BlockSpec((NBLK, 1), lambda: (0, 0)),
        ],
        out_shape=[
            jax.ShapeDtypeStruct((N_TOKENS, TOP_K), jnp.int32),
            jax.ShapeDtypeStruct((N_TOKENS, TOP_K), jnp.float32),
            jax.ShapeDtypeStruct((NBLK, 1), jnp.int32),
        ],
    )(tokens, Wr)


def _dispatch_sc_body(tokens_hbm, slot0_hbm, slot1_hbm, disp_hbm,
                      idx0_v, idx1_v, xbuf, sem):
    wid = lax.axis_index("s") * 2 + lax.axis_index("c")
    base = wid * TPW
    pltpu.sync_copy(slot0_hbm.at[pl.ds(base, TPW)], idx0_v)
    pltpu.sync_copy(slot1_hbm.at[pl.ds(base, TPW)], idx1_v)
    for ch in range(TPW // 16):
        pltpu.sync_copy(tokens_hbm.at[pl.ds(base + ch * 16, 16)], xbuf)
        i0 = idx0_v[pl.ds(ch * 16, 16)]
        pltpu.async_copy(xbuf, disp_hbm.at[i0], sem).wait()
        i1 = idx1_v[pl.ds(ch * 16, 16)]
        pltpu.async_copy(xbuf, disp_hbm.at[i1], sem).wait()


def _gmm_body(be_ref, x_ref, w1_ref, b1_ref, w2_ref, b2_ref, y_ref):
    del be_ref
    h = lax.dot_general(x_ref[...], w1_ref[0], (((1,), (1,)), ((), ())),
                        preferred_element_type=jnp.float32)
    h = _gelu_exact(h + b1_ref[0])
    y_ref[...] = lax.dot_general(h, w2_ref[0], (((1,), (1,)), ((), ())),
                                 preferred_element_type=jnp.float32) + b2_ref[0]


def _gmm(be, disp, W1, b1, W2, b2):
    grid_spec = pltpu.PrefetchScalarGridSpec(
        num_scalar_prefetch=1,
        grid=(NBLK,),
        in_specs=[
            pl.BlockSpec((BLK, HIDDEN), lambda i, be: (i, 0)),
            pl.BlockSpec((1, FF, HIDDEN), lambda i, be: (be[i], 0, 0)),
            pl.BlockSpec((1, 1, FF), lambda i, be: (be[i], 0, 0)),
            pl.BlockSpec((1, HIDDEN, FF), lambda i, be: (be[i], 0, 0)),
            pl.BlockSpec((1, 1, HIDDEN), lambda i, be: (be[i], 0, 0)),
        ],
        out_specs=pl.BlockSpec((BLK, HIDDEN), lambda i, be: (i, 0)),
    )
    return pl.pallas_call(
        _gmm_body,
        grid_spec=grid_spec,
        out_shape=jax.ShapeDtypeStruct((S_PAD, HIDDEN), jnp.float32),
    )(be, disp, W1, b1[:, None, :], W2, b2[:, None, :])


def _gather_sc_body(y_hbm, slot0_hbm, slot1_hbm, g0_hbm, g1_hbm,
                    idx0_v, idx1_v, rbuf, sem):
    wid = lax.axis_index("s") * 2 + lax.axis_index("c")
    base = wid * TPW
    pltpu.sync_copy(slot0_hbm.at[pl.ds(base, TPW)], idx0_v)
    pltpu.sync_copy(slot1_hbm.at[pl.ds(base, TPW)], idx1_v)
    for ch in range(TPW // 16):
        i0 = idx0_v[pl.ds(ch * 16, 16)]
        pltpu.async_copy(y_hbm.at[i0], rbuf, sem).wait()
        pltpu.sync_copy(rbuf, g0_hbm.at[pl.ds(base + ch * 16, 16)])
        i1 = idx1_v[pl.ds(ch * 16, 16)]
        pltpu.async_copy(y_hbm.at[i1], rbuf, sem).wait()
        pltpu.sync_copy(rbuf, g1_hbm.at[pl.ds(base + ch * 16, 16)])


@functools.lru_cache(maxsize=None)
def _sc_kernels():
    mesh = plsc.VectorSubcoreMesh(core_axis_name="c", subcore_axis_name="s")
    common_scratch = [
        pltpu.VMEM((TPW,), jnp.int32),
        pltpu.VMEM((TPW,), jnp.int32),
        pltpu.VMEM((16, HIDDEN), jnp.float32),
        pltpu.SemaphoreType.DMA,
    ]
    dispatch = pl.kernel(
        _dispatch_sc_body,
        out_type=jax.ShapeDtypeStruct((S_PAD, HIDDEN), jnp.float32),
        mesh=mesh,
        scratch_types=common_scratch,
    )
    gather = pl.kernel(
        _gather_sc_body,
        out_type=(jax.ShapeDtypeStruct((N_TOKENS, HIDDEN), jnp.float32),
                  jax.ShapeDtypeStruct((N_TOKENS, HIDDEN), jnp.float32)),
        mesh=mesh,
        scratch_types=common_scratch,
    )
    return dispatch, gather


def _comb_body(p_ref, g0_ref, g1_ref, out_ref):
    p0 = p_ref[:, 0:1]
    p1 = p_ref[:, 1:2]
    out_ref[...] = p0 * g0_ref[...] + p1 * g1_ref[...]


def _combine(p01, G0, G1):
    tb = 1024
    return pl.pallas_call(
        _comb_body,
        grid=(N_TOKENS // tb,),
        in_specs=[
            pl.BlockSpec((tb, TOP_K), lambda i: (i, 0)),
            pl.BlockSpec((tb, HIDDEN), lambda i: (i, 0)),
            pl.BlockSpec((tb, HIDDEN), lambda i: (i, 0)),
        ],
        out_specs=pl.BlockSpec((tb, HIDDEN), lambda i: (i, 0)),
        out_shape=jax.ShapeDtypeStruct((N_TOKENS, HIDDEN), jnp.float32),
    )(p01, G0, G1)


@jax.jit
def kernel(tokens, Wr, W1, b1, W2, b2):
    slot01, p01, be2 = _route_meta(tokens, Wr)
    slot0 = slot01[:, 0]
    slot1 = slot01[:, 1]
    be = be2.reshape(NBLK)
    dispatch_k, gather_k = _sc_kernels()
    disp = dispatch_k(tokens, slot0, slot1)
    Y = _gmm(be, disp, W1, b1, W2, b2)
    G0, G1 = gather_k(Y, slot0, slot1)
    return _combine(p01, G0, G1)
